# Initial kernel scaffold; baseline (speedup 1.0000x reference)
import functools

import jax
import jax.numpy as jnp
from jax import lax
from jax.experimental import pallas as pl
from jax.experimental.pallas import tpu as pltpu
from jax.experimental.pallas import tpu_sc as plsc

N = 10000
E = 320000
D = 128
HID = 128
C = 64
HD = D // 2

NC = 2
NT = 16
EPT = E // NT
KB = 80
NB = EPT // KB
RPT = N // NT
ZR = 25
DEG_CHUNK = 2 * E // (NC * NT)

_mesh = plsc.VectorSubcoreMesh(core_axis_name="c", subcore_axis_name="s")


@functools.partial(
    pl.kernel,
    mesh=_mesh,
    out_type=jax.ShapeDtypeStruct((NC * NT, N), jnp.float32),
    scratch_types=[
        pltpu.VMEM((DEG_CHUNK,), jnp.int32),
        pltpu.VMEM((N,), jnp.float32),
    ],
)
def _deg_kernel(idx_hbm, out_hbm, idxv, cnt):
    c = lax.axis_index("c")
    s = lax.axis_index("s")
    w = c * NT + s
    pltpu.sync_copy(idx_hbm.at[w], idxv)
    z16 = jnp.zeros((16,), jnp.float32)
    ones16 = jnp.ones((16,), jnp.float32)

    def zbody(i, carry):
        cnt[pl.ds(i * 16, 16)] = z16
        return carry

    lax.fori_loop(0, N // 16, zbody, 0)

    def body(i, carry):
        iv = idxv[pl.ds(i * 16, 16)]
        plsc.addupdate_scatter(cnt, [iv], ones16)
        return carry

    lax.fori_loop(0, DEG_CHUNK // 16, body, 0)
    pltpu.sync_copy(cnt, out_hbm.at[w])


@functools.partial(
    pl.kernel,
    mesh=_mesh,
    out_type=[
        jax.ShapeDtypeStruct((N, HD), jnp.float32),
        jax.ShapeDtypeStruct((N, HD), jnp.float32),
    ],
    scratch_types=[
        pltpu.VMEM((NB, KB), jnp.int32),
        pltpu.VMEM((NB, KB), jnp.int32),
        pltpu.VMEM((2, KB, HD), jnp.float32),
        pltpu.VMEM((ZR, HD), jnp.float32),
        pltpu.VMEM_SHARED((N, HD), jnp.float32),
        pltpu.SemaphoreType.DMA,
        pltpu.SemaphoreType.DMA,
    ],
)
def _prop_kernel(srcb, dstb, qlo, qhi, slo, shi,
                 srcv, dstv, rows, zv, acc, sem0, sem1):
    c = lax.axis_index("c")
    s = lax.axis_index("s")
    pltpu.sync_copy(srcb.at[s], srcv)
    pltpu.sync_copy(dstb.at[s], dstv)

    z16 = jnp.zeros((16,), jnp.float32)
    for r in range(ZR):
        for cc in range(HD // 16):
            zv[r, cc * 16:(cc + 1) * 16] = z16

    def zcopy(j, carry):
        pltpu.sync_copy(zv, acc.at[pl.ds(s * RPT + j * ZR, ZR)])
        return carry

    lax.fori_loop(0, RPT // ZR, zcopy, 0)
    plsc.subcore_barrier()

    def run_edges(q, out):
        sems = (sem0, sem1)

        def gather(b, slot, sem):
            return pltpu.make_async_copy(q.at[srcv.at[b]], rows.at[slot], sem)

        gather(0, 0, sem0).start()

        def outer(g, carry):
            for j in range(2):
                b = g * 2 + j
                nxt = b + 1

                @pl.when(nxt < NB)
                def _():
                    gather(nxt, 1 - j, sems[1 - j]).start()

                gather(b, j, sems[j]).wait()
                pltpu.sync_copy(rows.at[j], acc.at[dstv.at[b]], add=True)
            return carry

        lax.fori_loop(0, NB // 2, outer, 0)
        plsc.subcore_barrier()
        pltpu.sync_copy(acc.at[pl.ds(s * RPT, RPT)],
                        out.at[pl.ds(s * RPT, RPT)])

    @pl.when(c == 0)
    def _():
        run_edges(qlo, slo)

    @pl.when(c == 1)
    def _():
        run_edges(qhi, shi)


def _dinv_body(degp_ref, dinv_ref, dinv2_ref):
    deg = jnp.sum(degp_ref[...], axis=0)
    dv = lax.rsqrt(jnp.clip(deg, 1.0, None))
    dinv_ref[...] = dv
    dinv2_ref[...] = dv * dv


_dinv_kernel = pl.pallas_call(
    _dinv_body,
    out_shape=[jax.ShapeDtypeStruct((N,), jnp.float32)] * 2,
)

BR = 2000


def _scale_x_body(x_ref, dv_ref, qlo_ref, qhi_ref):
    q = x_ref[...] * dv_ref[...]
    qlo_ref[...] = q[:, :HD]
    qhi_ref[...] = q[:, HD:]


_scale_x = pl.pallas_call(
    _scale_x_body,
    grid=(N // BR,),
    in_specs=[
        pl.BlockSpec((BR, D), lambda i: (i, 0)),
        pl.BlockSpec((BR, 1), lambda i: (i, 0)),
    ],
    out_specs=[pl.BlockSpec((BR, HD), lambda i: (i, 0))] * 2,
    out_shape=[jax.ShapeDtypeStruct((N, HD), jnp.float32)] * 2,
)


def _scale_s_body(sl_ref, sh_ref, dv2_ref, qlo_ref, qhi_ref):
    dv2 = dv2_ref[...]
    qlo_ref[...] = sl_ref[...] * dv2
    qhi_ref[...] = sh_ref[...] * dv2


_scale_s = pl.pallas_call(
    _scale_s_body,
    grid=(N // BR,),
    in_specs=[
        pl.BlockSpec((BR, HD), lambda i: (i, 0)),
        pl.BlockSpec((BR, HD), lambda i: (i, 0)),
        pl.BlockSpec((BR, 1), lambda i: (i, 0)),
    ],
    out_specs=[pl.BlockSpec((BR, HD), lambda i: (i, 0))] * 2,
    out_shape=[jax.ShapeDtypeStruct((N, HD), jnp.float32)] * 2,
)


def _final_body(s1l, s1h, s2l, s2h, s3l, s3h, dv_ref,
                w1_ref, w2_ref, w3_ref, wd_ref, bd_ref, out_ref):
    dv = dv_ref[...]

    def head(sl, sh, w_ref, wd0):
        h = jnp.dot(sl[...] * dv, w_ref[:HD, :],
                    preferred_element_type=jnp.float32,
                    precision=lax.Precision.HIGHEST)
        h += jnp.dot(sh[...] * dv, w_ref[HD:, :],
                     preferred_element_type=jnp.float32,
                     precision=lax.Precision.HIGHEST)
        h = jnp.maximum(h, 0.0)
        return jnp.dot(h, wd_ref[wd0:wd0 + HID, :],
                       preferred_element_type=jnp.float32,
                       precision=lax.Precision.HIGHEST)

    out = head(s1l, s1h, w1_ref, 0)
    out += head(s2l, s2h, w2_ref, HID)
    out += head(s3l, s3h, w3_ref, 2 * HID)
    out_ref[...] = out + bd_ref[...]


_final_kernel = pl.pallas_call(
    _final_body,
    grid=(N // BR,),
    in_specs=[pl.BlockSpec((BR, HD), lambda i: (i, 0))] * 6 + [
        pl.BlockSpec((BR, 1), lambda i: (i, 0)),
        pl.BlockSpec((D, HID), lambda i: (0, 0)),
        pl.BlockSpec((D, HID), lambda i: (0, 0)),
        pl.BlockSpec((D, HID), lambda i: (0, 0)),
        pl.BlockSpec((3 * HID, C), lambda i: (0, 0)),
        pl.BlockSpec((1, C), lambda i: (0, 0)),
    ],
    out_specs=pl.BlockSpec((BR, C), lambda i: (i, 0)),
    out_shape=jax.ShapeDtypeStruct((N, C), jnp.float32),
)


def kernel(x, edge_index, W1, W2, W3, Wd, bd):
    src = edge_index[0]
    dst = edge_index[1]
    allidx = jnp.concatenate([src, dst]).reshape(NC * NT, DEG_CHUNK)
    srcb = src.reshape(NT, NB, KB)
    dstb = dst.reshape(NT, NB, KB)

    degp = _deg_kernel(allidx)
    dinv, dinv2 = _dinv_kernel(degp)
    dinv_c = dinv.reshape(N, 1)
    dinv2_c = dinv2.reshape(N, 1)

    q0l, q0h = _scale_x(x, dinv_c)
    s1l, s1h = _prop_kernel(srcb, dstb, q0l, q0h)
    q1l, q1h = _scale_s(s1l, s1h, dinv2_c)
    s2l, s2h = _prop_kernel(srcb, dstb, q1l, q1h)
    q2l, q2h = _scale_s(s2l, s2h, dinv2_c)
    s3l, s3h = _prop_kernel(srcb, dstb, q2l, q2h)

    return _final_kernel(s1l, s1h, s2l, s2h, s3l, s3h, dinv_c,
                         W1, W2, W3, Wd, bd.reshape(1, C))


# speedup vs baseline: 19.1931x; 19.1931x over previous
import functools

import jax
import jax.numpy as jnp
from jax import lax
from jax.experimental import pallas as pl
from jax.experimental.pallas import tpu as pltpu
from jax.experimental.pallas import tpu_sc as plsc

N = 10000
NP = 10240
E = 320000
D = 128
HID = 128
C = 64
HD = D // 2

NC = 2
NT = 16
EPT = E // NT
KB = 80
NB = EPT // KB
NCH = 5
CH = NB // NCH
RPT = NP // NT
ZR = 8
DEG_CHUNK = 2 * E // (NC * NT)

_mesh = plsc.VectorSubcoreMesh(core_axis_name="c", subcore_axis_name="s")


@functools.partial(
    pl.kernel,
    mesh=_mesh,
    out_type=jax.ShapeDtypeStruct((NC * NT * N,), jnp.float32),
    scratch_types=[
        pltpu.VMEM((DEG_CHUNK,), jnp.int32),
        pltpu.VMEM((N,), jnp.float32),
    ],
    compiler_params=pltpu.CompilerParams(needs_layout_passes=False),
)
def _deg_kernel(idx_hbm, out_hbm, idxv, cnt):
    c = lax.axis_index("c")
    s = lax.axis_index("s")
    w = c * NT + s
    pltpu.sync_copy(
        idx_hbm.at[pl.ds(pl.multiple_of(w * DEG_CHUNK, 8), DEG_CHUNK)], idxv)
    z16 = jnp.zeros((16,), jnp.float32)
    ones16 = jnp.ones((16,), jnp.float32)

    def zbody(i, carry):
        cnt[pl.ds(i * 16, 16)] = z16
        return carry

    lax.fori_loop(0, N // 16, zbody, 0)

    def body(i, carry):
        iv = idxv[pl.ds(i * 16, 16)]
        plsc.addupdate_scatter(cnt, [iv], ones16)
        return carry

    lax.fori_loop(0, DEG_CHUNK // 16, body, 0)
    pltpu.sync_copy(cnt, out_hbm.at[pl.ds(pl.multiple_of(w * N, 8), N)])


@functools.partial(
    pl.kernel,
    mesh=_mesh,
    out_type=[
        jax.ShapeDtypeStruct((NP, HD), jnp.float32),
        jax.ShapeDtypeStruct((NP, HD), jnp.float32),
    ],
    scratch_types=[
        pltpu.VMEM((CH, KB), jnp.int32),
        pltpu.VMEM((CH, KB), jnp.int32),
        pltpu.VMEM((2, KB, HD), jnp.float32),
        pltpu.VMEM((ZR, HD), jnp.float32),
        pltpu.VMEM_SHARED((NP, HD), jnp.float32),
        pltpu.SemaphoreType.DMA,
        pltpu.SemaphoreType.DMA,
    ],
    compiler_params=pltpu.CompilerParams(use_tc_tiling_on_sc=False),
)
def _prop_kernel(srcb, dstb, qlo, qhi, slo, shi,
                 srcv, dstv, rows, zv, acc, sem0, sem1):
    c = lax.axis_index("c")
    s = lax.axis_index("s")

    z16 = jnp.zeros((16,), jnp.float32)
    for r in range(ZR):
        for cc in range(HD // 16):
            zv[r, cc * 16:(cc + 1) * 16] = z16

    def zcopy(j, carry):
        pltpu.sync_copy(
            zv, acc.at[pl.ds(pl.multiple_of(s * RPT + j * ZR, 8), ZR)])
        return carry

    lax.fori_loop(0, RPT // ZR, zcopy, 0)
    plsc.subcore_barrier()

    sems = (sem0, sem1)

    def run_edges(q, out):
        def gather(b, slot):
            return pltpu.make_async_copy(
                q.at[srcv.at[b]], rows.at[slot], sems[slot])

        def chunk_body(ch, carry):
            cb = pl.multiple_of(ch * CH, 2)
            pltpu.sync_copy(srcb.at[s, pl.ds(cb, CH)], srcv)
            pltpu.sync_copy(dstb.at[s, pl.ds(cb, CH)], dstv)
            gather(0, 0).start()

            def pair_body(g, carry2):
                for j in range(2):
                    b = g * 2 + j
                    nxt = b + 1

                    @pl.when(nxt < CH)
                    def _():
                        gather(nxt, 1 - j).start()

                    gather(b, j).wait()
                    pltpu.sync_copy(rows.at[j], acc.at[dstv.at[b]], add=True)
                return carry2

            lax.fori_loop(0, CH // 2, pair_body, 0)
            return carry

        lax.fori_loop(0, NCH, chunk_body, 0)
        plsc.subcore_barrier()
        base = pl.multiple_of(s * RPT, 8)
        pltpu.sync_copy(acc.at[pl.ds(base, RPT)], out.at[pl.ds(base, RPT)])

    @pl.when(c == 0)
    def _():
        run_edges(qlo, slo)

    @pl.when(c == 1)
    def _():
        run_edges(qhi, shi)


def _dinv_body(degp_ref, dinv_ref, dinv2_ref):
    deg = jnp.sum(degp_ref[...], axis=0)
    dv = lax.rsqrt(jnp.clip(deg, 1.0, None))
    dinv_ref[...] = dv
    dinv2_ref[...] = dv * dv


_dinv_kernel = pl.pallas_call(
    _dinv_body,
    out_shape=[jax.ShapeDtypeStruct((N,), jnp.float32)] * 2,
)

BR = 2000


def _scale_x_body(x_ref, dv_ref, qlo_ref, qhi_ref):
    q = x_ref[...] * dv_ref[...]
    qlo_ref[...] = q[:, :HD]
    qhi_ref[...] = q[:, HD:]


_scale_x = pl.pallas_call(
    _scale_x_body,
    grid=(N // BR,),
    in_specs=[
        pl.BlockSpec((BR, D), lambda i: (i, 0)),
        pl.BlockSpec((BR, 1), lambda i: (i, 0)),
    ],
    out_specs=[pl.BlockSpec((BR, HD), lambda i: (i, 0))] * 2,
    out_shape=[jax.ShapeDtypeStruct((N, HD), jnp.float32)] * 2,
)


def _scale_s_body(sl_ref, sh_ref, dv2_ref, qlo_ref, qhi_ref):
    dv2 = dv2_ref[...]
    qlo_ref[...] = sl_ref[...] * dv2
    qhi_ref[...] = sh_ref[...] * dv2


_scale_s = pl.pallas_call(
    _scale_s_body,
    grid=(N // BR,),
    in_specs=[
        pl.BlockSpec((BR, HD), lambda i: (i, 0)),
        pl.BlockSpec((BR, HD), lambda i: (i, 0)),
        pl.BlockSpec((BR, 1), lambda i: (i, 0)),
    ],
    out_specs=[pl.BlockSpec((BR, HD), lambda i: (i, 0))] * 2,
    out_shape=[jax.ShapeDtypeStruct((N, HD), jnp.float32)] * 2,
)


def _final_body(s1l, s1h, s2l, s2h, s3l, s3h, dv_ref,
                w1_ref, w2_ref, w3_ref, wd_ref, bd_ref, out_ref):
    dv = dv_ref[...]

    def head(sl, sh, w_ref, wd0):
        h = jnp.dot(sl[...] * dv, w_ref[:HD, :],
                    preferred_element_type=jnp.float32,
                    precision=lax.Precision.HIGHEST)
        h += jnp.dot(sh[...] * dv, w_ref[HD:, :],
                     preferred_element_type=jnp.float32,
                     precision=lax.Precision.HIGHEST)
        h = jnp.maximum(h, 0.0)
        return jnp.dot(h, wd_ref[wd0:wd0 + HID, :],
                       preferred_element_type=jnp.float32,
                       precision=lax.Precision.HIGHEST)

    out = head(s1l, s1h, w1_ref, 0)
    out += head(s2l, s2h, w2_ref, HID)
    out += head(s3l, s3h, w3_ref, 2 * HID)
    out_ref[...] = out + bd_ref[...]


_final_kernel = pl.pallas_call(
    _final_body,
    grid=(N // BR,),
    in_specs=[pl.BlockSpec((BR, HD), lambda i: (i, 0))] * 6 + [
        pl.BlockSpec((BR, 1), lambda i: (i, 0)),
        pl.BlockSpec((D, HID), lambda i: (0, 0)),
        pl.BlockSpec((D, HID), lambda i: (0, 0)),
        pl.BlockSpec((D, HID), lambda i: (0, 0)),
        pl.BlockSpec((3 * HID, C), lambda i: (0, 0)),
        pl.BlockSpec((1, C), lambda i: (0, 0)),
    ],
    out_specs=pl.BlockSpec((BR, C), lambda i: (i, 0)),
    out_shape=jax.ShapeDtypeStruct((N, C), jnp.float32),
)


def kernel(x, edge_index, W1, W2, W3, Wd, bd):
    src = edge_index[0]
    dst = edge_index[1]
    allidx = jnp.concatenate([src, dst])
    srcb = src.reshape(NT, NB, KB)
    dstb = dst.reshape(NT, NB, KB)

    degp = _deg_kernel(allidx).reshape(NC * NT, N)
    dinv, dinv2 = _dinv_kernel(degp)
    dinv_c = dinv.reshape(N, 1)
    dinv2_c = dinv2.reshape(N, 1)

    q0l, q0h = _scale_x(x, dinv_c)
    s1l, s1h = _prop_kernel(srcb, dstb, q0l, q0h)
    q1l, q1h = _scale_s(s1l, s1h, dinv2_c)
    s2l, s2h = _prop_kernel(srcb, dstb, q1l, q1h)
    q2l, q2h = _scale_s(s2l, s2h, dinv2_c)
    s3l, s3h = _prop_kernel(srcb, dstb, q2l, q2h)

    return _final_kernel(s1l, s1h, s2l, s2h, s3l, s3h, dinv_c,
                         W1, W2, W3, Wd, bd.reshape(1, C))
